# Initial kernel scaffold; baseline (speedup 1.0000x reference)
import functools

import jax
import jax.numpy as jnp
from jax import lax
from jax.experimental import pallas as pl
from jax.experimental.pallas import tpu as pltpu
from jax.experimental.pallas import tpu_sc as plsc

_EMB_DIM = 1000000
_OUT_DIM = 32
_N_ROWS = 16384
_N_COLS = 200
_B = _N_ROWS * _N_COLS
_IW = 128
_B2 = _B // _IW
_NC = 2
_NS = 16
_NW = _NC * _NS
_RPW = _B2 // _NW
_K = 8
_CHUNKS = _RPW // _K


def _idx_body(x_ref, o_ref):
    xb = x_ref[...]
    s = jnp.sum(xb, axis=1, keepdims=True)
    o_ref[...] = ((xb / s) * float(_EMB_DIM - 1)).astype(jnp.int32)


def _compute_idx(x):
    blk = 2048
    return pl.pallas_call(
        _idx_body,
        grid=(_N_ROWS // blk,),
        in_specs=[pl.BlockSpec((blk, _N_COLS), lambda i: (i, 0))],
        out_specs=pl.BlockSpec((blk, _N_COLS), lambda i: (i, 0)),
        out_shape=jax.ShapeDtypeStruct((_N_ROWS, _N_COLS), jnp.int32),
    )(x)


def _ln_body(t_ref, g_ref, b_ref, o_ref):
    t = t_ref[...]
    mean = jnp.mean(t, axis=1, keepdims=True)
    var = jnp.mean(jnp.square(t - mean), axis=1, keepdims=True)
    o_ref[...] = (t - mean) / jnp.sqrt(var + 1e-5) * g_ref[...] + b_ref[...]


def _norm_table(table, gamma, beta):
    blk = 4000
    return pl.pallas_call(
        _ln_body,
        grid=(_EMB_DIM // blk,),
        in_specs=[
            pl.BlockSpec((blk, _OUT_DIM), lambda i: (i, 0)),
            pl.BlockSpec((1, _OUT_DIM), lambda i: (0, 0)),
            pl.BlockSpec((1, _OUT_DIM), lambda i: (0, 0)),
        ],
        out_specs=pl.BlockSpec((blk, _OUT_DIM), lambda i: (i, 0)),
        out_shape=jax.ShapeDtypeStruct((_EMB_DIM, _OUT_DIM), jnp.float32),
    )(table, gamma.reshape(1, _OUT_DIM), beta.reshape(1, _OUT_DIM))


def _gather_body(tbl_hbm, idx_hbm, out_hbm, idx_v, rows_v, gsem):
    c = lax.axis_index("c")
    s = lax.axis_index("s")
    wid = s * _NC + c
    base = wid * _RPW

    def chunk(g, carry):
        r0 = base + g * _K
        pltpu.sync_copy(idx_hbm.at[pl.ds(r0, _K)], idx_v)
        copies = [
            pltpu.async_copy(tbl_hbm.at[idx_v.at[j]], rows_v.at[j], gsem)
            for j in range(_K)
        ]
        for cp in copies:
            cp.wait()
        pltpu.sync_copy(rows_v, out_hbm.at[pl.ds(r0, _K)])
        return carry

    lax.fori_loop(0, _CHUNKS, chunk, 0)


def _gather(ntable, idx2d):
    mesh = plsc.VectorSubcoreMesh(core_axis_name="c", subcore_axis_name="s")
    fn = pl.kernel(
        _gather_body,
        mesh=mesh,
        out_type=jax.ShapeDtypeStruct((_B2, _IW, _OUT_DIM), jnp.float32),
        scratch_types=[
            pltpu.VMEM((_K, _IW), jnp.int32),
            pltpu.VMEM((_K, _IW, _OUT_DIM), jnp.float32),
            pltpu.SemaphoreType.DMA,
        ],
    )
    return fn(ntable, idx2d)


def kernel(x, table, ln_gamma, ln_beta):
    idx = _compute_idx(x).reshape(_B2, _IW)
    ntable = _norm_table(table, ln_gamma, ln_beta)
    out = _gather(ntable, idx)
    return out.reshape(_N_ROWS, _N_COLS, _OUT_DIM)


# speedup vs baseline: 3.9235x; 3.9235x over previous
import functools

import jax
import jax.numpy as jnp
from jax import lax
from jax.experimental import pallas as pl
from jax.experimental.pallas import tpu as pltpu
from jax.experimental.pallas import tpu_sc as plsc

_EMB_DIM = 1000000
_OUT_DIM = 32
_N_ROWS = 16384
_N_COLS = 200
_B = _N_ROWS * _N_COLS
_IW = 128
_B2 = _B // _IW
_NC = 2
_NS = 16
_NW = _NC * _NS
_RPW = _B2 // _NW
_K = 8
_CHUNKS = _RPW // _K


def _idx_body(x_ref, o_ref):
    xb = x_ref[...]
    acc = xb[:, 0:8]
    for k in range(1, 25):
        acc = acc + xb[:, 8 * k:8 * k + 8]
    t1 = acc[:, 0:4] + acc[:, 4:8]
    t2 = t1[:, 0:2] + t1[:, 2:4]
    s = t2[:, 0:1] + t2[:, 1:2]
    o_ref[...] = ((xb / s) * float(_EMB_DIM - 1)).astype(jnp.int32)


def _compute_idx(x):
    blk = 2048
    return pl.pallas_call(
        _idx_body,
        grid=(_N_ROWS // blk,),
        in_specs=[pl.BlockSpec((blk, _N_COLS), lambda i: (i, 0))],
        out_specs=pl.BlockSpec((blk, _N_COLS), lambda i: (i, 0)),
        out_shape=jax.ShapeDtypeStruct((_N_ROWS, _N_COLS), jnp.int32),
    )(x)


def _ln_body(t_ref, g_ref, b_ref, o_ref):
    t = t_ref[...]
    mean = jnp.mean(t, axis=1, keepdims=True)
    var = jnp.mean(jnp.square(t - mean), axis=1, keepdims=True)
    o_ref[...] = (t - mean) / jnp.sqrt(var + 1e-5) * g_ref[...] + b_ref[...]


def _norm_table(table, gamma, beta):
    blk = 4000
    return pl.pallas_call(
        _ln_body,
        grid=(_EMB_DIM // blk,),
        in_specs=[
            pl.BlockSpec((blk, _OUT_DIM), lambda i: (i, 0)),
            pl.BlockSpec((1, _OUT_DIM), lambda i: (0, 0)),
            pl.BlockSpec((1, _OUT_DIM), lambda i: (0, 0)),
        ],
        out_specs=pl.BlockSpec((blk, _OUT_DIM), lambda i: (i, 0)),
        out_shape=jax.ShapeDtypeStruct((_EMB_DIM, _OUT_DIM), jnp.float32),
    )(table, gamma.reshape(1, _OUT_DIM), beta.reshape(1, _OUT_DIM))


def _gather_body(tbl_hbm, idx_hbm, out_hbm, idx_v, rows_v, gsem):
    c = lax.axis_index("c")
    s = lax.axis_index("s")
    wid = s * _NC + c
    base = wid * _RPW

    def chunk(g, carry):
        r0 = base + g * _K
        pltpu.sync_copy(idx_hbm.at[pl.ds(r0, _K)], idx_v)
        copies = [
            pltpu.async_copy(tbl_hbm.at[idx_v.at[j]], rows_v.at[j], gsem)
            for j in range(_K)
        ]
        for cp in copies:
            cp.wait()
        pltpu.sync_copy(rows_v, out_hbm.at[pl.ds(r0, _K)])
        return carry

    lax.fori_loop(0, _CHUNKS, chunk, 0)


def _gather(ntable, idx2d):
    mesh = plsc.VectorSubcoreMesh(core_axis_name="c", subcore_axis_name="s")
    fn = pl.kernel(
        _gather_body,
        mesh=mesh,
        out_type=jax.ShapeDtypeStruct((_B2, _IW, _OUT_DIM), jnp.float32),
        scratch_types=[
            pltpu.VMEM((_K, _IW), jnp.int32),
            pltpu.VMEM((_K, _IW, _OUT_DIM), jnp.float32),
            pltpu.SemaphoreType.DMA,
        ],
        compiler_params=pltpu.CompilerParams(use_tc_tiling_on_sc=False),
    )
    return fn(ntable, idx2d)


def kernel(x, table, ln_gamma, ln_beta):
    idx = _compute_idx(x).reshape(_B2, _IW)
    ntable = _norm_table(table, ln_gamma, ln_beta)
    out = _gather(ntable, idx)
    return out.reshape(_N_ROWS, _N_COLS, _OUT_DIM)


# speedup vs baseline: 4.4753x; 1.1407x over previous
import functools

import jax
import jax.numpy as jnp
from jax import lax
from jax.experimental import pallas as pl
from jax.experimental.pallas import tpu as pltpu
from jax.experimental.pallas import tpu_sc as plsc

_EMB_DIM = 1000000
_OUT_DIM = 32
_N_ROWS = 16384
_N_COLS = 200
_B = _N_ROWS * _N_COLS
_IW = 128
_B2 = _B // _IW
_NC = 2
_NS = 16
_NW = _NC * _NS
_RPW = _B2 // _NW
_K = 8
_CHUNKS = _RPW // _K


def _idx_body_t(xT_ref, o_ref):
    xT = xT_ref[...]
    acc = xT[0:8, :]
    for k in range(1, 25):
        acc = acc + xT[8 * k:8 * k + 8, :]
    t1 = acc[0:4, :] + acc[4:8, :]
    t2 = t1[0:2, :] + t1[2:4, :]
    s = t2[0:1, :] + t2[1:2, :]
    idxT = ((xT / s) * float(_EMB_DIM - 1)).astype(jnp.int32)
    o_ref[...] = idxT.T


def _compute_idx(x):
    blk = 2048
    return pl.pallas_call(
        _idx_body_t,
        grid=(_N_ROWS // blk,),
        in_specs=[pl.BlockSpec((_N_COLS, blk), lambda i: (0, i))],
        out_specs=pl.BlockSpec((blk, _N_COLS), lambda i: (i, 0)),
        out_shape=jax.ShapeDtypeStruct((_N_ROWS, _N_COLS), jnp.int32),
    )(x.T)


def _ln_body_t(tT_ref, g_ref, b_ref, o_ref):
    t = tT_ref[...]
    mean = jnp.mean(t, axis=0, keepdims=True)
    var = jnp.mean(jnp.square(t - mean), axis=0, keepdims=True)
    normed = (t - mean) / jnp.sqrt(var + 1e-5)
    out = normed * g_ref[...] + b_ref[...]
    o_ref[...] = out.T


def _norm_table(table, gamma, beta):
    blk = 2048
    grid = (_EMB_DIM + blk - 1) // blk
    return pl.pallas_call(
        _ln_body_t,
        grid=(grid,),
        in_specs=[
            pl.BlockSpec((_OUT_DIM, blk), lambda i: (0, i)),
            pl.BlockSpec((_OUT_DIM, 1), lambda i: (0, 0)),
            pl.BlockSpec((_OUT_DIM, 1), lambda i: (0, 0)),
        ],
        out_specs=pl.BlockSpec((blk, _OUT_DIM), lambda i: (i, 0)),
        out_shape=jax.ShapeDtypeStruct((_EMB_DIM, _OUT_DIM), jnp.float32),
    )(table.T, gamma.reshape(_OUT_DIM, 1), beta.reshape(_OUT_DIM, 1))


def _gather_body(tbl_hbm, idx_hbm, out_hbm, idx_v, rows_v, gsem):
    c = lax.axis_index("c")
    s = lax.axis_index("s")
    wid = s * _NC + c
    base = wid * _RPW

    def chunk(g, carry):
        r0 = base + g * _K
        pltpu.sync_copy(idx_hbm.at[pl.ds(r0, _K)], idx_v)
        copies = [
            pltpu.async_copy(tbl_hbm.at[idx_v.at[j]], rows_v.at[j], gsem)
            for j in range(_K)
        ]
        for cp in copies:
            cp.wait()
        pltpu.sync_copy(rows_v, out_hbm.at[pl.ds(r0, _K)])
        return carry

    lax.fori_loop(0, _CHUNKS, chunk, 0)


def _gather(ntable, idx2d):
    mesh = plsc.VectorSubcoreMesh(core_axis_name="c", subcore_axis_name="s")
    fn = pl.kernel(
        _gather_body,
        mesh=mesh,
        out_type=jax.ShapeDtypeStruct((_B2, _IW, _OUT_DIM), jnp.float32),
        scratch_types=[
            pltpu.VMEM((_K, _IW), jnp.int32),
            pltpu.VMEM((_K, _IW, _OUT_DIM), jnp.float32),
            pltpu.SemaphoreType.DMA,
        ],
        compiler_params=pltpu.CompilerParams(use_tc_tiling_on_sc=False),
    )
    return fn(ntable, idx2d)


def kernel(x, table, ln_gamma, ln_beta):
    idx = _compute_idx(x).reshape(_B2, _IW)
    ntable = _norm_table(table, ln_gamma, ln_beta)
    out = _gather(ntable, idx)
    return out.reshape(_N_ROWS, _N_COLS, _OUT_DIM)
